# baseline (speedup 1.0000x reference)
import jax
import jax.numpy as jnp
from jax.experimental import pallas as pl
from jax.experimental.pallas import tpu as pltpu

MARGIN = 0.3
PD_EPS = 1e-6
B = 1024
D = 2048
BM = 256
NI = B // BM
PC = 256
NC = B // PC


def _triplet_kernel(x1_ref, x2_ref, out_ref,
                    pbf_ref, t2_ref, colp_ref,
                    araw_ref, praw_ref, asem, psem):
    i = pl.program_id(0)
    ones_row = jnp.ones((1, D), jnp.float32)

    @pl.when(i == 0)
    def _init():
        pltpu.make_async_copy(
            x1_ref.at[pl.ds(0, BM), 0, :],
            araw_ref.at[0], asem.at[0]).start()
        for c in range(NC):
            pltpu.make_async_copy(
                x2_ref.at[pl.ds(c * PC, PC), 1, :],
                praw_ref.at[c], psem.at[c]).start()
        for k in range(1, NI):
            pltpu.make_async_copy(
                x1_ref.at[pl.ds(k * BM, BM), 0, :],
                araw_ref.at[k], asem.at[k]).start()
        out_ref[...] = jnp.zeros_like(out_ref)
        for c in range(NC):
            pltpu.make_async_copy(
                x2_ref.at[pl.ds(c * PC, PC), 1, :],
                praw_ref.at[c], psem.at[c]).wait()
            praw = praw_ref[c]
            pbf_ref[c * PC:(c + 1) * PC, :] = praw.astype(jnp.bfloat16)
            np2 = jax.lax.dot_general(
                praw * praw, ones_row, (((1,), (1,)), ((), ())),
                preferred_element_type=jnp.float32)
            sump = jax.lax.dot_general(
                praw, ones_row, (((1,), (1,)), ((), ())),
                preferred_element_type=jnp.float32)
            t = 1.0 / jnp.maximum(jnp.sqrt(np2), 1e-12)
            t2_ref[c * PC:(c + 1) * PC, :] = 2.0 * t
            colp_ref[c * PC:(c + 1) * PC, :] = np2 * t * t - (2.0 * PD_EPS) * sump * t

    pltpu.make_async_copy(
        x1_ref.at[pl.ds(i * BM, BM), 0, :],
        araw_ref.at[i], asem.at[i]).wait()
    a = araw_ref[i]
    abf = a.astype(jnp.bfloat16)
    na2 = jax.lax.dot_general(
        ones_row, a * a, (((1,), (1,)), ((), ())),
        preferred_element_type=jnp.float32)
    sa = 1.0 / jnp.maximum(jnp.sqrt(na2), 1e-12)

    cross = jax.lax.dot_general(
        pbf_ref[...], abf, (((1,), (1,)), ((), ())),
        preferred_element_type=jnp.float32)
    h = colp_ref[...] - (t2_ref[...] * cross) * sa

    rowj = jax.lax.broadcasted_iota(jnp.int32, (B, BM), 0)
    coli = jax.lax.broadcasted_iota(jnp.int32, (B, BM), 1) + i * BM
    diag = rowj == coli

    hneg = jnp.min(jnp.where(diag, jnp.float32(3.0e38), h), axis=0,
                   keepdims=True)
    hpos = jnp.sum(jnp.where(diag, h, 0.0), axis=0, keepdims=True)
    lv = jnp.maximum(hpos - hneg + MARGIN, 0.0) * (1.0 / B)
    out_ref[...] += jnp.sum(lv, axis=1, keepdims=True)


def kernel(x1, x2):
    out = pl.pallas_call(
        _triplet_kernel,
        grid=(NI,),
        in_specs=[
            pl.BlockSpec(memory_space=pltpu.HBM),
            pl.BlockSpec(memory_space=pltpu.HBM),
        ],
        out_specs=pl.BlockSpec((1, 1), lambda i: (0, 0)),
        out_shape=jax.ShapeDtypeStruct((1, 1), jnp.float32),
        scratch_shapes=[
            pltpu.VMEM((B, D), jnp.bfloat16),
            pltpu.VMEM((B, 1), jnp.float32),
            pltpu.VMEM((B, 1), jnp.float32),
            pltpu.VMEM((NI, BM, D), jnp.float32),
            pltpu.VMEM((NC, PC, D), jnp.float32),
            pltpu.SemaphoreType.DMA((NI,)),
            pltpu.SemaphoreType.DMA((NC,)),
        ],
        compiler_params=pltpu.CompilerParams(
            dimension_semantics=("arbitrary",),
        ),
    )(x1, x2)
    return out[0, 0]
